# baseline (device time: 227523 ns/iter reference)
import functools

import jax
import jax.numpy as jnp
from jax import lax
from jax.experimental import pallas as pl
from jax.experimental.pallas import tpu as pltpu

NZ = 4
NXY = 4
M_CHUNK = 512
F_BLK = 2048
H = 4


def _body(xt_ref, dyb_ref, out_ref, recv_buf, sbuf, pbuf,
          send1, recv1, send2, recv2):
    mx = lax.axis_index("x")
    my = lax.axis_index("y")
    mz = lax.axis_index("z")
    fb = 2 * mx + my

    z_next = (mz + 1) % NZ
    z_prev = (mz - 1) % NZ

    xy_peers = [
        (mx, 1 - my),
        (1 - mx, my),
        (1 - mx, 1 - my),
    ]

    neighbors = [
        (mx, my, z_next),
        (mx, my, z_prev),
    ] + [(px, py, mz) for px, py in xy_peers]

    barrier_sem = pltpu.get_barrier_semaphore()
    for nbr in neighbors:
        pl.semaphore_signal(
            barrier_sem, inc=1, device_id=nbr,
            device_id_type=pl.DeviceIdType.MESH,
        )
    pl.semaphore_wait(barrier_sem, len(neighbors))

    blk = F_BLK // H
    d1 = {}
    d2 = {}
    send_waited = set()

    def pblock(row_idx, c):
        return jnp.dot(
            xt_ref[pl.ds(row_idx * M_CHUNK, M_CHUNK), :],
            dyb_ref[:, pl.ds(c * blk, blk)],
            preferred_element_type=jnp.float32,
        )

    def p1_send(c, s):
        row = (mz - 1 - s) % NZ
        if s == 0:
            pbuf[:, pl.ds(c * blk, blk)] = pblock(row, c)
            src = pbuf.at[:, pl.ds(c * blk, blk)]
        else:
            if s >= 2:
                d1[(c, s - 1)].wait_send()
                send_waited.add((c, s - 1))
            sbuf[:, pl.ds(c * blk, blk)] = (
                recv_buf[s - 1, :, pl.ds(c * blk, blk)] + pblock(row, c)
            )
            src = sbuf.at[:, pl.ds(c * blk, blk)]
        rdma = pltpu.make_async_remote_copy(
            src_ref=src,
            dst_ref=recv_buf.at[s, :, pl.ds(c * blk, blk)],
            send_sem=send1.at[s, c],
            recv_sem=recv1.at[s, c],
            device_id=(mx, my, z_next),
            device_id_type=pl.DeviceIdType.MESH,
        )
        rdma.start()
        d1[(c, s)] = rdma

    def p1_recv(c, s):
        d1[(c, s)].wait_recv()

    def final(c):
        p1_recv(c, NZ - 2)
        col0 = fb * F_BLK + c * blk
        out_ref[:, pl.ds(col0, blk)] = (
            recv_buf[NZ - 2, :, pl.ds(c * blk, blk)] + pblock(mz, c)
        )
        for j, (px, py) in enumerate(xy_peers):
            rdma = pltpu.make_async_remote_copy(
                src_ref=out_ref.at[:, pl.ds(col0, blk)],
                dst_ref=out_ref.at[:, pl.ds(col0, blk)],
                send_sem=send2.at[j, c],
                recv_sem=recv2.at[j, c],
                device_id=(px, py, mz),
                device_id_type=pl.DeviceIdType.MESH,
            )
            rdma.start()
            d2[(j, c)] = rdma

    p1_send(0, 0)
    p1_send(1, 0)
    p1_recv(0, 0)
    p1_send(0, 1)
    p1_send(2, 0)
    p1_recv(0, 1)
    p1_send(0, 2)
    p1_send(3, 0)
    p1_recv(1, 0)
    p1_send(1, 1)
    final(0)
    p1_recv(1, 1)
    p1_send(1, 2)
    p1_recv(2, 0)
    p1_send(2, 1)
    final(1)
    p1_recv(2, 1)
    p1_send(2, 2)
    p1_recv(3, 0)
    p1_send(3, 1)
    final(2)
    p1_recv(3, 1)
    p1_send(3, 2)
    final(3)

    for j in range(len(xy_peers)):
        for c in range(H):
            d2[(j, c)].wait_recv()
    for key, rdma in d1.items():
        if key not in send_waited:
            rdma.wait_send()
    for rdma in d2.values():
        rdma.wait_send()

    @functools.partial(pl.run_scoped, sem=pltpu.SemaphoreType.REGULAR)
    def _(sem):
        for nbr in neighbors:
            pl.semaphore_signal(
                sem, inc=1, device_id=nbr,
                device_id_type=pl.DeviceIdType.MESH,
            )
        pl.semaphore_wait(sem, len(neighbors))


def kernel(x, dy):
    mx = lax.axis_index("x")
    my = lax.axis_index("y")
    fb = 2 * mx + my

    k, m = x.shape
    _, f = dy.shape
    f_blk = f // NXY

    xt_bf = lax.transpose(x, (1, 0)).astype(jnp.bfloat16)
    dyb_bf = lax.dynamic_slice(
        dy, (0, fb * f_blk), (k, f_blk)
    ).astype(jnp.bfloat16)

    m_chunk = m // NZ

    return pl.pallas_call(
        _body,
        out_shape=jax.ShapeDtypeStruct((m_chunk, f), jnp.float32),
        in_specs=[
            pl.BlockSpec(memory_space=pltpu.VMEM),
            pl.BlockSpec(memory_space=pltpu.VMEM),
        ],
        out_specs=pl.BlockSpec(memory_space=pltpu.VMEM),
        scratch_shapes=[
            pltpu.VMEM((NZ - 1, m_chunk, f_blk), jnp.float32),
            pltpu.VMEM((m_chunk, f_blk), jnp.float32),
            pltpu.VMEM((m_chunk, f_blk), jnp.float32),
            pltpu.SemaphoreType.DMA((NZ - 1, H)),
            pltpu.SemaphoreType.DMA((NZ - 1, H)),
            pltpu.SemaphoreType.DMA((NXY - 1, H)),
            pltpu.SemaphoreType.DMA((NXY - 1, H)),
        ],
        compiler_params=pltpu.CompilerParams(
            collective_id=0,
            vmem_limit_bytes=100 * 1024 * 1024,
        ),
    )(xt_bf, dyb_bf)


# device time: 222054 ns/iter; 1.0246x vs baseline; 1.0246x over previous
import functools

import jax
import jax.numpy as jnp
from jax import lax
from jax.experimental import pallas as pl
from jax.experimental.pallas import tpu as pltpu

NZ = 4
NXY = 4
M_CHUNK = 512
F_BLK = 2048
H = 4


def _body(xt_ref, dyb_ref, out_ref, recv_buf, pcache,
          send1, recv1, send2, recv2):
    mx = lax.axis_index("x")
    my = lax.axis_index("y")
    mz = lax.axis_index("z")
    fb = 2 * mx + my

    z_next = (mz + 1) % NZ
    z_prev = (mz - 1) % NZ

    xy_peers = [
        (mx, 1 - my),
        (1 - mx, my),
        (1 - mx, 1 - my),
    ]

    neighbors = [
        (mx, my, z_next),
        (mx, my, z_prev),
    ] + [(px, py, mz) for px, py in xy_peers]

    barrier_sem = pltpu.get_barrier_semaphore()
    for nbr in neighbors:
        pl.semaphore_signal(
            barrier_sem, inc=1, device_id=nbr,
            device_id_type=pl.DeviceIdType.MESH,
        )
    pl.semaphore_wait(barrier_sem, len(neighbors))

    blk = F_BLK // H
    d1 = {}
    d2 = {}

    rows = [(mz - 1) % NZ, (mz - 2) % NZ, (mz - 3) % NZ, mz]

    def dot_into(s, c):
        row = rows[s]
        pcache[row, :, pl.ds(c * blk, blk)] = jnp.dot(
            xt_ref[pl.ds(row * M_CHUNK, M_CHUNK), :],
            dyb_ref[:, pl.ds(c * blk, blk)],
            preferred_element_type=jnp.float32,
        )

    def p1_send(c, s):
        row = rows[s]
        if s > 0:
            pcache[row, :, pl.ds(c * blk, blk)] = (
                pcache[row, :, pl.ds(c * blk, blk)]
                + recv_buf[s - 1, :, pl.ds(c * blk, blk)]
            )
        rdma = pltpu.make_async_remote_copy(
            src_ref=pcache.at[row, :, pl.ds(c * blk, blk)],
            dst_ref=recv_buf.at[s, :, pl.ds(c * blk, blk)],
            send_sem=send1.at[s, c],
            recv_sem=recv1.at[s, c],
            device_id=(mx, my, z_next),
            device_id_type=pl.DeviceIdType.MESH,
        )
        rdma.start()
        d1[(c, s)] = rdma

    def p1_recv(c, s):
        d1[(c, s)].wait_recv()

    def final(c):
        p1_recv(c, NZ - 2)
        col0 = fb * F_BLK + c * blk
        out_ref[:, pl.ds(col0, blk)] = (
            recv_buf[NZ - 2, :, pl.ds(c * blk, blk)]
            + pcache[rows[3], :, pl.ds(c * blk, blk)]
        )
        for j, (px, py) in enumerate(xy_peers):
            rdma = pltpu.make_async_remote_copy(
                src_ref=out_ref.at[:, pl.ds(col0, blk)],
                dst_ref=out_ref.at[:, pl.ds(col0, blk)],
                send_sem=send2.at[j, c],
                recv_sem=recv2.at[j, c],
                device_id=(px, py, mz),
                device_id_type=pl.DeviceIdType.MESH,
            )
            rdma.start()
            d2[(j, c)] = rdma

    dot_into(0, 0)
    p1_send(0, 0)
    dot_into(0, 1)
    p1_send(1, 0)
    dot_into(0, 2)
    dot_into(0, 3)
    dot_into(1, 0)
    p1_recv(0, 0)
    p1_send(0, 1)
    p1_send(2, 0)
    dot_into(1, 1)
    dot_into(2, 0)
    p1_recv(0, 1)
    p1_send(0, 2)
    p1_send(3, 0)
    p1_recv(1, 0)
    p1_send(1, 1)
    dot_into(3, 0)
    dot_into(1, 2)
    dot_into(2, 1)
    final(0)
    dot_into(3, 1)
    dot_into(2, 2)
    p1_recv(1, 1)
    p1_send(1, 2)
    p1_recv(2, 0)
    p1_send(2, 1)
    final(1)
    dot_into(1, 3)
    dot_into(3, 2)
    p1_recv(2, 1)
    p1_send(2, 2)
    p1_recv(3, 0)
    p1_send(3, 1)
    final(2)
    dot_into(2, 3)
    dot_into(3, 3)
    p1_recv(3, 1)
    p1_send(3, 2)
    final(3)

    for j in range(len(xy_peers)):
        for c in range(H):
            d2[(j, c)].wait_recv()
    for rdma in d1.values():
        rdma.wait_send()
    for rdma in d2.values():
        rdma.wait_send()

    @functools.partial(pl.run_scoped, sem=pltpu.SemaphoreType.REGULAR)
    def _(sem):
        for nbr in neighbors:
            pl.semaphore_signal(
                sem, inc=1, device_id=nbr,
                device_id_type=pl.DeviceIdType.MESH,
            )
        pl.semaphore_wait(sem, len(neighbors))


def kernel(x, dy):
    mx = lax.axis_index("x")
    my = lax.axis_index("y")
    fb = 2 * mx + my

    k, m = x.shape
    _, f = dy.shape
    f_blk = f // NXY

    xt_bf = lax.transpose(x, (1, 0)).astype(jnp.bfloat16)
    dyb_bf = lax.dynamic_slice(
        dy, (0, fb * f_blk), (k, f_blk)
    ).astype(jnp.bfloat16)

    m_chunk = m // NZ

    return pl.pallas_call(
        _body,
        out_shape=jax.ShapeDtypeStruct((m_chunk, f), jnp.float32),
        in_specs=[
            pl.BlockSpec(memory_space=pltpu.VMEM),
            pl.BlockSpec(memory_space=pltpu.VMEM),
        ],
        out_specs=pl.BlockSpec(memory_space=pltpu.VMEM),
        scratch_shapes=[
            pltpu.VMEM((NZ - 1, m_chunk, f_blk), jnp.float32),
            pltpu.VMEM((NZ, m_chunk, f_blk), jnp.float32),
            pltpu.SemaphoreType.DMA((NZ - 1, H)),
            pltpu.SemaphoreType.DMA((NZ - 1, H)),
            pltpu.SemaphoreType.DMA((NXY - 1, H)),
            pltpu.SemaphoreType.DMA((NXY - 1, H)),
        ],
        compiler_params=pltpu.CompilerParams(
            collective_id=0,
            vmem_limit_bytes=100 * 1024 * 1024,
        ),
    )(xt_bf, dyb_bf)


# device time: 150690 ns/iter; 1.5099x vs baseline; 1.4736x over previous
import functools

import jax
import jax.numpy as jnp
from jax import lax
from jax.experimental import pallas as pl
from jax.experimental.pallas import tpu as pltpu

NZ = 4
NXY = 4
M_CHUNK = 512
F_BLK = 2048
H = 4


def _body(xt_ref, dyb_ref, out_ref, pcache, recv_bf, stage_bf, s_bf,
          ag_bf, send1, recv1, send2, recv2):
    mx = lax.axis_index("x")
    my = lax.axis_index("y")
    mz = lax.axis_index("z")
    fb = 2 * mx + my

    z_next = (mz + 1) % NZ
    z_prev = (mz - 1) % NZ

    xy_peers = [
        (mx, 1 - my),
        (1 - mx, my),
        (1 - mx, 1 - my),
    ]

    neighbors = [
        (mx, my, z_next),
        (mx, my, z_prev),
    ] + [(px, py, mz) for px, py in xy_peers]

    barrier_sem = pltpu.get_barrier_semaphore()
    for nbr in neighbors:
        pl.semaphore_signal(
            barrier_sem, inc=1, device_id=nbr,
            device_id_type=pl.DeviceIdType.MESH,
        )
    pl.semaphore_wait(barrier_sem, len(neighbors))

    blk = F_BLK // H
    d1 = {}
    d2 = {}

    rows = [(mz - 1) % NZ, (mz - 2) % NZ, (mz - 3) % NZ, mz]

    def cs(c):
        return pl.ds(c * blk, blk)

    def dot_strip(c):
        pcache[:, cs(c)] = jnp.dot(
            xt_ref[...], dyb_ref[:, cs(c)],
            preferred_element_type=jnp.float32,
        ).astype(jnp.bfloat16)

    def p1_send(c, s):
        rd = pl.ds(rows[s] * M_CHUNK, M_CHUNK)
        if s == 0:
            src = pcache.at[rd, cs(c)]
        else:
            stage_bf[s - 1, :, cs(c)] = (
                recv_bf[s - 1, :, cs(c)].astype(jnp.float32)
                + pcache[rd, cs(c)].astype(jnp.float32)
            ).astype(jnp.bfloat16)
            src = stage_bf.at[s - 1, :, cs(c)]
        rdma = pltpu.make_async_remote_copy(
            src_ref=src,
            dst_ref=recv_bf.at[s, :, cs(c)],
            send_sem=send1.at[s, c],
            recv_sem=recv1.at[s, c],
            device_id=(mx, my, z_next),
            device_id_type=pl.DeviceIdType.MESH,
        )
        rdma.start()
        d1[(c, s)] = rdma

    def p1_recv(c, s):
        d1[(c, s)].wait_recv()

    def final(c):
        p1_recv(c, NZ - 2)
        s_f32 = (
            recv_bf[NZ - 2, :, cs(c)].astype(jnp.float32)
            + pcache[pl.ds(rows[3] * M_CHUNK, M_CHUNK), cs(c)].astype(
                jnp.float32
            )
        )
        out_ref[:, pl.ds(fb * F_BLK + c * blk, blk)] = s_f32
        s_bf[:, cs(c)] = s_f32.astype(jnp.bfloat16)
        for j, (px, py) in enumerate(xy_peers):
            rdma = pltpu.make_async_remote_copy(
                src_ref=s_bf.at[:, cs(c)],
                dst_ref=ag_bf.at[j, :, cs(c)],
                send_sem=send2.at[j, c],
                recv_sem=recv2.at[j, c],
                device_id=(px, py, mz),
                device_id_type=pl.DeviceIdType.MESH,
            )
            rdma.start()
            d2[(j, c)] = rdma

    dot_strip(0)
    p1_send(0, 0)
    dot_strip(1)
    p1_send(1, 0)
    dot_strip(2)
    dot_strip(3)
    p1_recv(0, 0)
    p1_send(0, 1)
    p1_send(2, 0)
    p1_recv(0, 1)
    p1_send(0, 2)
    p1_send(3, 0)
    p1_recv(1, 0)
    p1_send(1, 1)
    final(0)
    p1_recv(1, 1)
    p1_send(1, 2)
    p1_recv(2, 0)
    p1_send(2, 1)
    final(1)
    p1_recv(2, 1)
    p1_send(2, 2)
    p1_recv(3, 0)
    p1_send(3, 1)
    final(2)
    p1_recv(3, 1)
    p1_send(3, 2)
    final(3)

    for c in range(H):
        for j, (px, py) in enumerate(xy_peers):
            d2[(j, c)].wait_recv()
            peer_fb = 2 * px + py
            out_ref[:, pl.ds(peer_fb * F_BLK + c * blk, blk)] = (
                ag_bf[j, :, cs(c)].astype(jnp.float32)
            )

    for rdma in d1.values():
        rdma.wait_send()
    for rdma in d2.values():
        rdma.wait_send()

    @functools.partial(pl.run_scoped, sem=pltpu.SemaphoreType.REGULAR)
    def _(sem):
        for nbr in neighbors:
            pl.semaphore_signal(
                sem, inc=1, device_id=nbr,
                device_id_type=pl.DeviceIdType.MESH,
            )
        pl.semaphore_wait(sem, len(neighbors))


def kernel(x, dy):
    mx = lax.axis_index("x")
    my = lax.axis_index("y")
    fb = 2 * mx + my

    k, m = x.shape
    _, f = dy.shape
    f_blk = f // NXY

    xt_bf = lax.transpose(x, (1, 0)).astype(jnp.bfloat16)
    dyb_bf = lax.dynamic_slice(
        dy, (0, fb * f_blk), (k, f_blk)
    ).astype(jnp.bfloat16)

    m_chunk = m // NZ

    return pl.pallas_call(
        _body,
        out_shape=jax.ShapeDtypeStruct((m_chunk, f), jnp.float32),
        in_specs=[
            pl.BlockSpec(memory_space=pltpu.VMEM),
            pl.BlockSpec(memory_space=pltpu.VMEM),
        ],
        out_specs=pl.BlockSpec(memory_space=pltpu.VMEM),
        scratch_shapes=[
            pltpu.VMEM((m, f_blk), jnp.bfloat16),
            pltpu.VMEM((NZ - 1, m_chunk, f_blk), jnp.bfloat16),
            pltpu.VMEM((NZ - 2, m_chunk, f_blk), jnp.bfloat16),
            pltpu.VMEM((m_chunk, f_blk), jnp.bfloat16),
            pltpu.VMEM((NXY - 1, m_chunk, f_blk), jnp.bfloat16),
            pltpu.SemaphoreType.DMA((NZ - 1, H)),
            pltpu.SemaphoreType.DMA((NZ - 1, H)),
            pltpu.SemaphoreType.DMA((NXY - 1, H)),
            pltpu.SemaphoreType.DMA((NXY - 1, H)),
        ],
        compiler_params=pltpu.CompilerParams(
            collective_id=0,
            vmem_limit_bytes=100 * 1024 * 1024,
        ),
    )(xt_bf, dyb_bf)


# device time: 143491 ns/iter; 1.5856x vs baseline; 1.0502x over previous
import functools

import jax
import jax.numpy as jnp
from jax import lax
from jax.experimental import pallas as pl
from jax.experimental.pallas import tpu as pltpu

NZ = 4
NXY = 4
M_CHUNK = 512
F_BLK = 2048
H = 4


def _body(xt_ref, dyb_ref, out_ref, pcache, recv_bf, stage_bf, s_bf,
          ag_bf, send1, recv1, send2, recv2):
    mx = lax.axis_index("x")
    my = lax.axis_index("y")
    mz = lax.axis_index("z")
    fb = 2 * mx + my

    z_next = (mz + 1) % NZ
    z_prev = (mz - 1) % NZ

    xy_peers = [
        (mx, 1 - my),
        (1 - mx, my),
        (1 - mx, 1 - my),
    ]

    neighbors = [
        (mx, my, z_next),
        (mx, my, z_prev),
    ] + [(px, py, mz) for px, py in xy_peers]

    barrier_sem = pltpu.get_barrier_semaphore()
    for nbr in neighbors:
        pl.semaphore_signal(
            barrier_sem, inc=1, device_id=nbr,
            device_id_type=pl.DeviceIdType.MESH,
        )
    pl.semaphore_wait(barrier_sem, len(neighbors))

    blk = F_BLK // H
    d1 = {}
    d2 = {}

    rows = [(mz - 1) % NZ, (mz - 2) % NZ, (mz - 3) % NZ, mz]

    def cs(c):
        return pl.ds(c * blk, blk)

    def dot_strip(c):
        pcache[:, cs(c)] = jnp.dot(
            xt_ref[...], dyb_ref[:, cs(c)],
            preferred_element_type=jnp.float32,
        ).astype(jnp.bfloat16)

    def subdot(c, s):
        rd = pl.ds(rows[s] * M_CHUNK, M_CHUNK)
        pcache[rd, cs(c)] = jnp.dot(
            xt_ref[rd, :], dyb_ref[:, cs(c)],
            preferred_element_type=jnp.float32,
        ).astype(jnp.bfloat16)

    def p1_send(c, s):
        rd = pl.ds(rows[s] * M_CHUNK, M_CHUNK)
        if s == 0:
            src = pcache.at[rd, cs(c)]
        else:
            stage_bf[s - 1, :, cs(c)] = (
                recv_bf[s - 1, :, cs(c)].astype(jnp.float32)
                + pcache[rd, cs(c)].astype(jnp.float32)
            ).astype(jnp.bfloat16)
            src = stage_bf.at[s - 1, :, cs(c)]
        rdma = pltpu.make_async_remote_copy(
            src_ref=src,
            dst_ref=recv_bf.at[s, :, cs(c)],
            send_sem=send1.at[s, c],
            recv_sem=recv1.at[s, c],
            device_id=(mx, my, z_next),
            device_id_type=pl.DeviceIdType.MESH,
        )
        rdma.start()
        d1[(c, s)] = rdma

    def p1_recv(c, s):
        d1[(c, s)].wait_recv()

    def final(c):
        p1_recv(c, NZ - 2)
        s_f32 = (
            recv_bf[NZ - 2, :, cs(c)].astype(jnp.float32)
            + pcache[pl.ds(rows[3] * M_CHUNK, M_CHUNK), cs(c)].astype(
                jnp.float32
            )
        )
        out_ref[:, pl.ds(fb * F_BLK + c * blk, blk)] = s_f32
        s_bf[:, cs(c)] = s_f32.astype(jnp.bfloat16)
        for j, (px, py) in enumerate(xy_peers):
            rdma = pltpu.make_async_remote_copy(
                src_ref=s_bf.at[:, cs(c)],
                dst_ref=ag_bf.at[j, :, cs(c)],
                send_sem=send2.at[j, c],
                recv_sem=recv2.at[j, c],
                device_id=(px, py, mz),
                device_id_type=pl.DeviceIdType.MESH,
            )
            rdma.start()
            d2[(j, c)] = rdma

    subdot(0, 0)
    p1_send(0, 0)
    subdot(1, 0)
    p1_send(1, 0)
    subdot(0, 1)
    subdot(1, 1)
    dot_strip(2)
    p1_recv(0, 0)
    p1_send(0, 1)
    p1_send(2, 0)
    subdot(0, 2)
    dot_strip(3)
    p1_recv(0, 1)
    p1_send(0, 2)
    p1_send(3, 0)
    p1_recv(1, 0)
    p1_send(1, 1)
    subdot(0, 3)
    subdot(1, 2)
    final(0)
    subdot(1, 3)
    p1_recv(1, 1)
    p1_send(1, 2)
    p1_recv(2, 0)
    p1_send(2, 1)
    final(1)
    p1_recv(2, 1)
    p1_send(2, 2)
    p1_recv(3, 0)
    p1_send(3, 1)
    final(2)
    p1_recv(3, 1)
    p1_send(3, 2)
    final(3)

    for c in range(H):
        for j, (px, py) in enumerate(xy_peers):
            d2[(j, c)].wait_recv()
            peer_fb = 2 * px + py
            out_ref[:, pl.ds(peer_fb * F_BLK + c * blk, blk)] = (
                ag_bf[j, :, cs(c)].astype(jnp.float32)
            )

    for rdma in d1.values():
        rdma.wait_send()
    for rdma in d2.values():
        rdma.wait_send()

    @functools.partial(pl.run_scoped, sem=pltpu.SemaphoreType.REGULAR)
    def _(sem):
        for nbr in neighbors:
            pl.semaphore_signal(
                sem, inc=1, device_id=nbr,
                device_id_type=pl.DeviceIdType.MESH,
            )
        pl.semaphore_wait(sem, len(neighbors))


def kernel(x, dy):
    mx = lax.axis_index("x")
    my = lax.axis_index("y")
    fb = 2 * mx + my

    k, m = x.shape
    _, f = dy.shape
    f_blk = f // NXY

    xt_bf = lax.transpose(x, (1, 0)).astype(jnp.bfloat16)
    dyb_bf = lax.dynamic_slice(
        dy, (0, fb * f_blk), (k, f_blk)
    ).astype(jnp.bfloat16)

    m_chunk = m // NZ

    return pl.pallas_call(
        _body,
        out_shape=jax.ShapeDtypeStruct((m_chunk, f), jnp.float32),
        in_specs=[
            pl.BlockSpec(memory_space=pltpu.VMEM),
            pl.BlockSpec(memory_space=pltpu.VMEM),
        ],
        out_specs=pl.BlockSpec(memory_space=pltpu.VMEM),
        scratch_shapes=[
            pltpu.VMEM((m, f_blk), jnp.bfloat16),
            pltpu.VMEM((NZ - 1, m_chunk, f_blk), jnp.bfloat16),
            pltpu.VMEM((NZ - 2, m_chunk, f_blk), jnp.bfloat16),
            pltpu.VMEM((m_chunk, f_blk), jnp.bfloat16),
            pltpu.VMEM((NXY - 1, m_chunk, f_blk), jnp.bfloat16),
            pltpu.SemaphoreType.DMA((NZ - 1, H)),
            pltpu.SemaphoreType.DMA((NZ - 1, H)),
            pltpu.SemaphoreType.DMA((NXY - 1, H)),
            pltpu.SemaphoreType.DMA((NXY - 1, H)),
        ],
        compiler_params=pltpu.CompilerParams(
            collective_id=0,
            vmem_limit_bytes=100 * 1024 * 1024,
        ),
    )(xt_bf, dyb_bf)


# device time: 138734 ns/iter; 1.6400x vs baseline; 1.0343x over previous
import functools

import jax
import jax.numpy as jnp
from jax import lax
from jax.experimental import pallas as pl
from jax.experimental.pallas import tpu as pltpu

NZ = 4
NXY = 4
M_CHUNK = 512
F_BLK = 2048
H = 4


def _body(xt_ref, dyb_ref, out_ref, pcache, recv_bf, stage_bf, s_bf,
          ag_bf, send1, recv1, send2, recv2):
    mx = lax.axis_index("x")
    my = lax.axis_index("y")
    mz = lax.axis_index("z")
    fb = 2 * mx + my

    z_next = (mz + 1) % NZ
    z_prev = (mz - 1) % NZ

    xy_peers = [
        (mx, 1 - my),
        (1 - mx, my),
        (1 - mx, 1 - my),
    ]

    neighbors = [
        (mx, my, z_next),
        (mx, my, z_prev),
    ] + [(px, py, mz) for px, py in xy_peers]

    barrier_sem = pltpu.get_barrier_semaphore()
    for nbr in neighbors:
        pl.semaphore_signal(
            barrier_sem, inc=1, device_id=nbr,
            device_id_type=pl.DeviceIdType.MESH,
        )
    pl.semaphore_wait(barrier_sem, len(neighbors))

    blk = F_BLK // H
    d1 = {}
    d2 = {}

    rows = [(mz - 1) % NZ, (mz - 2) % NZ, (mz - 3) % NZ, mz]

    def cs(c):
        return pl.ds(c * blk, blk)

    _DN = (((0,), (0,)), ((), ()))

    def dot_strip(c):
        pcache[:, cs(c)] = lax.dot_general(
            xt_ref[...], dyb_ref[:, cs(c)], _DN,
            preferred_element_type=jnp.float32,
        ).astype(jnp.bfloat16)

    def subdot(c, s):
        rd = pl.ds(rows[s] * M_CHUNK, M_CHUNK)
        pcache[rd, cs(c)] = lax.dot_general(
            xt_ref[:, rd], dyb_ref[:, cs(c)], _DN,
            preferred_element_type=jnp.float32,
        ).astype(jnp.bfloat16)

    def p1_send(c, s):
        rd = pl.ds(rows[s] * M_CHUNK, M_CHUNK)
        if s == 0:
            src = pcache.at[rd, cs(c)]
        else:
            stage_bf[s - 1, :, cs(c)] = (
                recv_bf[s - 1, :, cs(c)].astype(jnp.float32)
                + pcache[rd, cs(c)].astype(jnp.float32)
            ).astype(jnp.bfloat16)
            src = stage_bf.at[s - 1, :, cs(c)]
        rdma = pltpu.make_async_remote_copy(
            src_ref=src,
            dst_ref=recv_bf.at[s, :, cs(c)],
            send_sem=send1.at[s, c],
            recv_sem=recv1.at[s, c],
            device_id=(mx, my, z_next),
            device_id_type=pl.DeviceIdType.MESH,
        )
        rdma.start()
        d1[(c, s)] = rdma

    def p1_recv(c, s):
        d1[(c, s)].wait_recv()

    def final(c):
        p1_recv(c, NZ - 2)
        s_f32 = (
            recv_bf[NZ - 2, :, cs(c)].astype(jnp.float32)
            + pcache[pl.ds(rows[3] * M_CHUNK, M_CHUNK), cs(c)].astype(
                jnp.float32
            )
        )
        out_ref[:, pl.ds(fb * F_BLK + c * blk, blk)] = s_f32
        s_bf[:, cs(c)] = s_f32.astype(jnp.bfloat16)
        for j, (px, py) in enumerate(xy_peers):
            rdma = pltpu.make_async_remote_copy(
                src_ref=s_bf.at[:, cs(c)],
                dst_ref=ag_bf.at[j, :, cs(c)],
                send_sem=send2.at[j, c],
                recv_sem=recv2.at[j, c],
                device_id=(px, py, mz),
                device_id_type=pl.DeviceIdType.MESH,
            )
            rdma.start()
            d2[(j, c)] = rdma

    subdot(0, 0)
    p1_send(0, 0)
    subdot(1, 0)
    p1_send(1, 0)
    subdot(0, 1)
    subdot(1, 1)
    dot_strip(2)
    p1_recv(0, 0)
    p1_send(0, 1)
    p1_send(2, 0)
    subdot(0, 2)
    dot_strip(3)
    p1_recv(0, 1)
    p1_send(0, 2)
    p1_send(3, 0)
    p1_recv(1, 0)
    p1_send(1, 1)
    subdot(0, 3)
    subdot(1, 2)
    final(0)
    subdot(1, 3)
    p1_recv(1, 1)
    p1_send(1, 2)
    p1_recv(2, 0)
    p1_send(2, 1)
    final(1)
    p1_recv(2, 1)
    p1_send(2, 2)
    p1_recv(3, 0)
    p1_send(3, 1)
    final(2)
    p1_recv(3, 1)
    p1_send(3, 2)
    final(3)

    for c in range(H):
        for j, (px, py) in enumerate(xy_peers):
            d2[(j, c)].wait_recv()
            peer_fb = 2 * px + py
            out_ref[:, pl.ds(peer_fb * F_BLK + c * blk, blk)] = (
                ag_bf[j, :, cs(c)].astype(jnp.float32)
            )

    for rdma in d1.values():
        rdma.wait_send()
    for rdma in d2.values():
        rdma.wait_send()

    @functools.partial(pl.run_scoped, sem=pltpu.SemaphoreType.REGULAR)
    def _(sem):
        for nbr in neighbors:
            pl.semaphore_signal(
                sem, inc=1, device_id=nbr,
                device_id_type=pl.DeviceIdType.MESH,
            )
        pl.semaphore_wait(sem, len(neighbors))


def kernel(x, dy):
    mx = lax.axis_index("x")
    my = lax.axis_index("y")
    fb = 2 * mx + my

    k, m = x.shape
    _, f = dy.shape
    f_blk = f // NXY

    xt_bf = x.astype(jnp.bfloat16)
    dyb_bf = lax.dynamic_slice(
        dy, (0, fb * f_blk), (k, f_blk)
    ).astype(jnp.bfloat16)

    m_chunk = m // NZ

    return pl.pallas_call(
        _body,
        out_shape=jax.ShapeDtypeStruct((m_chunk, f), jnp.float32),
        in_specs=[
            pl.BlockSpec(memory_space=pltpu.VMEM),
            pl.BlockSpec(memory_space=pltpu.VMEM),
        ],
        out_specs=pl.BlockSpec(memory_space=pltpu.VMEM),
        scratch_shapes=[
            pltpu.VMEM((m, f_blk), jnp.bfloat16),
            pltpu.VMEM((NZ - 1, m_chunk, f_blk), jnp.bfloat16),
            pltpu.VMEM((NZ - 2, m_chunk, f_blk), jnp.bfloat16),
            pltpu.VMEM((m_chunk, f_blk), jnp.bfloat16),
            pltpu.VMEM((NXY - 1, m_chunk, f_blk), jnp.bfloat16),
            pltpu.SemaphoreType.DMA((NZ - 1, H)),
            pltpu.SemaphoreType.DMA((NZ - 1, H)),
            pltpu.SemaphoreType.DMA((NXY - 1, H)),
            pltpu.SemaphoreType.DMA((NXY - 1, H)),
        ],
        compiler_params=pltpu.CompilerParams(
            collective_id=0,
            vmem_limit_bytes=100 * 1024 * 1024,
        ),
    )(xt_bf, dyb_bf)
